# baseline (speedup 1.0000x reference)
import functools

import jax
import jax.numpy as jnp
from jax import lax
from jax.experimental import pallas as pl
from jax.experimental.pallas import tpu as pltpu
from jax.experimental.pallas import tpu_sc as plsc

_N_OUT = 4096
_N_IN = 4096
_D = 4
_K = 256
_NB = _N_IN // _D
_NC = 2
_NS = 16
_NW = _NC * _NS
_ROWS_PER_W = _N_OUT // _NW
_R_CHUNK = 8
_N_CHUNKS = _ROWS_PER_W // _R_CHUNK
_L = 16

_mesh = plsc.VectorSubcoreMesh(core_axis_name="c", subcore_axis_name="s")


@functools.partial(
    pl.kernel,
    mesh=_mesh,
    out_type=jax.ShapeDtypeStruct((_N_OUT, _N_IN), jnp.float32),
    compiler_params=pltpu.CompilerParams(needs_layout_passes=False),
    scratch_types=[
        [pltpu.VMEM((_K * _L,), jnp.float32) for _ in range(_D)],
        [pltpu.VMEM((_K * _L,), jnp.float32) for _ in range(2)],
        pltpu.VMEM((_D * _NB,), jnp.float32),
        pltpu.VMEM((_ROWS_PER_W,), jnp.float32),
        [pltpu.VMEM((_R_CHUNK, _NB), jnp.int32) for _ in range(2)],
        [pltpu.VMEM((_R_CHUNK, _N_IN), jnp.float32) for _ in range(2)],
        [pltpu.SemaphoreType.DMA for _ in range(2)],
        [pltpu.SemaphoreType.DMA for _ in range(2)],
    ],
)
def _sc_quantize(tab_hbm, rn_hbm, asn_hbm, cn_hbm, out_hbm,
                 tab_v, ptab_v, rn_v, cn_v, asn_v, out_v, in_sems, out_sems):
    wid = lax.axis_index("s") * _NC + lax.axis_index("c")
    row_base = wid * _ROWS_PER_W
    for d in range(_D):
        pltpu.sync_copy(tab_hbm.at[pl.ds(d * _K * _L, _K * _L)], tab_v[d])
    pltpu.sync_copy(rn_hbm, rn_v)
    pltpu.sync_copy(cn_hbm.at[pl.ds(row_base, _ROWS_PER_W)], cn_v)
    iota = lax.iota(jnp.int32, _L)

    def build_body(j, carry):
        for t in range(2):
            v0 = tab_v[2 * t][pl.ds(j * _L, _L)]
            v1 = tab_v[2 * t + 1][pl.ds(j * _L, _L)]
            pk = plsc.pack(v0, v1, format=plsc.PackFormat.INTERLEAVED)
            ptab_v[t][pl.ds(j * _L, _L)] = plsc.bitcast(pk, jnp.float32)
        return carry

    lax.fori_loop(0, _K, build_body, 0)

    def asn_copy(ci, h):
        r0 = row_base + ci * _R_CHUNK
        return pltpu.make_async_copy(
            asn_hbm.at[pl.ds(r0, _R_CHUNK)], asn_v[h], in_sems[h])

    def out_copy(ci, h):
        r0 = row_base + ci * _R_CHUNK
        return pltpu.make_async_copy(
            out_v[h], out_hbm.at[pl.ds(r0, _R_CHUNK)], out_sems[h])

    asn_copy(0, 0).start()

    def pair_body(p, carry):
        for h in range(2):
            ci = 2 * p + h
            asn_copy(ci, h).wait()

            @pl.when(ci + 1 < _N_CHUNKS)
            def _():
                asn_copy(ci + 1, 1 - h).start()

            @pl.when(ci >= 2)
            def _():
                out_copy(ci - 2, h).wait()

            cns = [plsc.load_gather(
                cn_v, [jnp.full((_L,), ci * _R_CHUNK + r, jnp.int32)])
                for r in range(_R_CHUNK)]
            rsplats = [jnp.full((_L,), r, jnp.int32) for r in range(_R_CHUNK)]

            @plsc.parallel_loop(0, _NB // _L, unroll=2)
            def b_body(b0):
                pos = [_D * _L * b0 + _D * iota + d for d in range(_D)]
                rns = [rn_v[pl.ds(d * _NB + b0 * _L, _L)] for d in range(_D)]
                for rr in range(0, _R_CHUNK, 4):
                    idxs = [asn_v[h][rr + j, pl.ds(b0 * _L, _L)] * _L + iota
                            for j in range(4)]
                    gps = [plsc.load_gather(ptab_v[t], [idxs[j]])
                           for j in range(4) for t in range(2)]
                    vals = []
                    for j in range(4):
                        for t in range(2):
                            g0, g1 = plsc.unpack(
                                plsc.bitcast(gps[j * 2 + t], jnp.bfloat16),
                                format=plsc.PackFormat.INTERLEAVED,
                                preferred_element_type=jnp.float32)
                            vals.append(g0 * rns[2 * t] * cns[rr + j])
                            vals.append(g1 * rns[2 * t + 1] * cns[rr + j])
                    for j in range(4):
                        for d in range(_D):
                            plsc.store_scatter(
                                out_v[h], [rsplats[rr + j], pos[d]],
                                vals[j * _D + d])

            out_copy(ci, h).start()
        return carry

    lax.fori_loop(0, _N_CHUNKS // 2, pair_body, 0)
    for h in range(2):
        out_copy(_N_CHUNKS - 2 + h, h).wait()


def kernel(centriods, assignments, rowwise_norms, columnwise_norms):
    tab = jnp.broadcast_to(
        centriods.astype(jnp.float32).T[:, :, None], (_D, _K, _L)).reshape(-1)
    rn = rowwise_norms.astype(jnp.float32).reshape(_NB, _D).T.reshape(-1)
    asn = assignments.astype(jnp.int32)
    cn = columnwise_norms.astype(jnp.float32)
    return _sc_quantize(tab, rn, asn, cn)


# speedup vs baseline: 1.2766x; 1.2766x over previous
import functools

import jax
import jax.numpy as jnp
from jax import lax
from jax.experimental import pallas as pl
from jax.experimental.pallas import tpu as pltpu
from jax.experimental.pallas import tpu_sc as plsc

_N_OUT = 4096
_N_IN = 4096
_D = 4
_K = 256
_NB = _N_IN // _D
_NC = 2
_NS = 16
_NW = _NC * _NS
_ROWS_PER_W = _N_OUT // _NW
_R_CHUNK = 8
_N_CHUNKS = _ROWS_PER_W // _R_CHUNK
_L = 16

_mesh = plsc.VectorSubcoreMesh(core_axis_name="c", subcore_axis_name="s")


@functools.partial(
    pl.kernel,
    mesh=_mesh,
    out_type=jax.ShapeDtypeStruct((_N_OUT, _N_IN), jnp.float32),
    compiler_params=pltpu.CompilerParams(needs_layout_passes=False),
    scratch_types=[
        [pltpu.VMEM((_K * _L,), jnp.float32) for _ in range(_D)],
        pltpu.VMEM((_D * _NB,), jnp.float32),
        pltpu.VMEM((_ROWS_PER_W,), jnp.float32),
        [pltpu.VMEM((_R_CHUNK, _NB), jnp.int32) for _ in range(2)],
        [pltpu.VMEM((_R_CHUNK, _N_IN), jnp.float32) for _ in range(2)],
        [pltpu.SemaphoreType.DMA for _ in range(2)],
        [pltpu.SemaphoreType.DMA for _ in range(2)],
    ],
)
def _sc_quantize(tab_hbm, rn_hbm, asn_hbm, cn_hbm, out_hbm,
                 tab_v, rn_v, cn_v, asn_v, out_v, in_sems, out_sems):
    wid = lax.axis_index("s") * _NC + lax.axis_index("c")
    row_base = wid * _ROWS_PER_W
    for d in range(_D):
        pltpu.sync_copy(tab_hbm.at[pl.ds(d * _K * _L, _K * _L)], tab_v[d])
    pltpu.sync_copy(rn_hbm, rn_v)
    pltpu.sync_copy(cn_hbm.at[pl.ds(row_base, _ROWS_PER_W)], cn_v)
    iota = lax.iota(jnp.int32, _L)

    def asn_copy(ci, h):
        r0 = row_base + ci * _R_CHUNK
        return pltpu.make_async_copy(
            asn_hbm.at[pl.ds(r0, _R_CHUNK)], asn_v[h], in_sems[h])

    def out_copy(ci, h):
        r0 = row_base + ci * _R_CHUNK
        return pltpu.make_async_copy(
            out_v[h], out_hbm.at[pl.ds(r0, _R_CHUNK)], out_sems[h])

    asn_copy(0, 0).start()

    def pair_body(p, carry):
        for h in range(2):
            ci = 2 * p + h
            asn_copy(ci, h).wait()

            @pl.when(ci + 1 < _N_CHUNKS)
            def _():
                asn_copy(ci + 1, 1 - h).start()

            @pl.when(ci >= 2)
            def _():
                out_copy(ci - 2, h).wait()

            cns = [plsc.load_gather(
                cn_v, [jnp.full((_L,), ci * _R_CHUNK + r, jnp.int32)])
                for r in range(_R_CHUNK)]
            rsplats = [jnp.full((_L,), r, jnp.int32) for r in range(_R_CHUNK)]

            @plsc.parallel_loop(0, _NB // _L, unroll=2)
            def b_body(b0):
                pos = [_D * _L * b0 + _D * iota + d for d in range(_D)]
                rns = [rn_v[pl.ds(d * _NB + b0 * _L, _L)] for d in range(_D)]
                for rr in range(0, _R_CHUNK, 4):
                    idxs = [asn_v[h][rr + j, pl.ds(b0 * _L, _L)] * _L + iota
                            for j in range(4)]
                    gs = [plsc.load_gather(tab_v[d], [idxs[j]])
                          for j in range(4) for d in range(_D)]
                    vals = [gs[j * _D + d] * rns[d] * cns[rr + j]
                            for j in range(4) for d in range(_D)]
                    for j in range(4):
                        for d in range(_D):
                            plsc.store_scatter(
                                out_v[h], [rsplats[rr + j], pos[d]],
                                vals[j * _D + d])

            out_copy(ci, h).start()
        return carry

    lax.fori_loop(0, _N_CHUNKS // 2, pair_body, 0)
    for h in range(2):
        out_copy(_N_CHUNKS - 2 + h, h).wait()


def kernel(centriods, assignments, rowwise_norms, columnwise_norms):
    tab = jnp.broadcast_to(
        centriods.astype(jnp.float32).T[:, :, None], (_D, _K, _L)).reshape(-1)
    rn = rowwise_norms.astype(jnp.float32).reshape(_NB, _D).T.reshape(-1)
    asn = assignments.astype(jnp.int32)
    cn = columnwise_norms.astype(jnp.float32)
    return _sc_quantize(tab, rn, asn, cn)
